# baseline (speedup 1.0000x reference)
import functools

import jax
import jax.numpy as jnp
from jax import lax
from jax.experimental import pallas as pl
from jax.experimental.pallas import tpu as pltpu
from jax.experimental.pallas import tpu_sc as plsc

_L = 16


@functools.partial(jax.jit, static_argnames=("d",))
def _gather_sc(vp, idx, d):
    b, h = idx.shape
    dp = vp.shape[1]
    info = plsc.get_sparse_core_info()
    nc = info.num_cores
    nw = nc * info.num_subcores
    tbs = b // dp
    tb_per_w = tbs // nw
    npan = tb_per_w * h
    te = d // 8

    mesh = plsc.VectorSubcoreMesh(core_axis_name="c", subcore_axis_name="s")

    @functools.partial(
        pl.kernel,
        mesh=mesh,
        out_type=jax.ShapeDtypeStruct((h, te, tbs, 8, dp), jnp.float32),
        compiler_params=pltpu.CompilerParams(use_tc_tiling_on_sc=False, needs_layout_passes=False),
        scratch_types=[
            pltpu.VMEM((dp, h), jnp.int32),
            pltpu.VMEM((2, dp), jnp.int32),
            pltpu.VMEM((2, dp, dp), jnp.float32),
            pltpu.VMEM((2, te, 8, dp), jnp.float32),
            pltpu.SemaphoreType.DMA,
            pltpu.SemaphoreType.DMA,
            pltpu.SemaphoreType.DMA,
            pltpu.SemaphoreType.DMA,
        ],
    )
    def k(table_hbm, idx_hbm, out_hbm, idx_blk, idx_list, rows_p, panel,
          gsem0, gsem1, wsem0, wsem1):
        wid = lax.axis_index("s") * nc + lax.axis_index("c")
        tb0 = wid * tb_per_w
        gsem = (gsem0, gsem1)
        wsem = (wsem0, wsem1)

        def load_blk(tb_local):
            pltpu.sync_copy(
                idx_hbm.at[pl.ds((tb0 + tb_local) * dp, dp)], idx_blk)

        def build_idx(hh, slot):
            lane = lax.iota(jnp.int32, _L)
            hv = jnp.full((_L,), hh, jnp.int32)
            for j0 in range(0, dp, _L):
                vals = plsc.load_gather(idx_blk, [j0 + lane, hv])
                idx_list[slot, pl.ds(j0, _L)] = vals

        def start_gather(slot):
            pltpu.make_async_copy(
                table_hbm.at[idx_list.at[slot]],
                rows_p.at[slot],
                gsem[slot],
            ).start()

        def wait_gather(slot):
            pltpu.make_async_copy(
                table_hbm.at[idx_list.at[slot]],
                rows_p.at[slot],
                gsem[slot],
            ).wait()

        def transpose(slot):
            src = rows_p.at[slot]

            def body(i, carry):
                lane = lax.iota(jnp.int32, _L)
                for t in range(te):
                    ev = jnp.full((_L,), 8 * t + i, jnp.int32)
                    for j0 in range(0, dp, _L):
                        vals = plsc.load_gather(src, [j0 + lane, ev])
                        panel[slot, t, i, pl.ds(j0, _L)] = vals
                return carry

            lax.fori_loop(0, 8, body, 0)

        def start_write(hh, tb_local, slot):
            pltpu.make_async_copy(
                panel.at[slot],
                out_hbm.at[hh, :, tb0 + tb_local],
                wsem[slot],
            ).start()

        def wait_write(slot):
            pltpu.make_async_copy(
                panel.at[slot],
                out_hbm.at[0, :, 0],
                wsem[slot],
            ).wait()

        def hh_of(p):
            return lax.rem(p, h)

        def tb_of(p):
            return lax.div(p, h)

        def step(p_dyn, slot, build_next, reload_pred):
            if reload_pred is not False:
                @pl.when(reload_pred)
                def _():
                    load_blk(tb_of(p_dyn) + 1)

            if build_next:
                build_idx(hh_of(p_dyn + 1), 1 - slot)
                start_gather(1 - slot)

            wait_gather(slot)
            transpose(slot)
            start_write(hh_of(p_dyn), tb_of(p_dyn), slot)

        load_blk(0)
        build_idx(0, 0)
        start_gather(0)
        step(jnp.int32(0), 0, True, False)
        step(jnp.int32(1), 1, True, False)

        def body(i, carry):
            a = 2 * i
            wait_write(0)
            step(a, 0, True, False)
            wait_write(1)
            step(a + 1, 1, True, hh_of(a + 1) == h - 1)
            return carry

        lax.fori_loop(1, npan // 2 - 1, body, 0)

        a = npan - 2
        wait_write(0)
        step(jnp.int32(a), 0, True, False)
        wait_write(1)
        step(jnp.int32(a + 1), 1, False, False)
        wait_write(0)
        wait_write(1)

    return k(vp, idx)


def kernel(v, idx):
    b, h = idx.shape
    d = v.shape[1]
    vp = jnp.pad(v, ((0, 0), (0, 128 - d)))
    y5 = _gather_sc(vp, idx, d)
    return y5.transpose(2, 4, 0, 1, 3).reshape(b, h, d)


# speedup vs baseline: 1.0018x; 1.0018x over previous
import functools

import jax
import jax.numpy as jnp
from jax import lax
from jax.experimental import pallas as pl
from jax.experimental.pallas import tpu as pltpu
from jax.experimental.pallas import tpu_sc as plsc

_L = 16


@functools.partial(jax.jit, static_argnames=("d",))
def _gather_sc(vp, idx, d):
    b, h = idx.shape
    dp = vp.shape[1]
    info = plsc.get_sparse_core_info()
    nc = info.num_cores
    nw = nc * info.num_subcores
    tbs = b // dp
    tb_per_w = tbs // nw
    npan = tb_per_w * h
    te = d // 8

    mesh = plsc.VectorSubcoreMesh(core_axis_name="c", subcore_axis_name="s")

    @functools.partial(
        pl.kernel,
        mesh=mesh,
        out_type=jax.ShapeDtypeStruct((h, te, tbs, 8, dp), jnp.float32),
        compiler_params=pltpu.CompilerParams(use_tc_tiling_on_sc=False, needs_layout_passes=False),
        scratch_types=[
            pltpu.VMEM((dp, h), jnp.int32),
            pltpu.VMEM((4, dp), jnp.int32),
            pltpu.VMEM((4, dp, dp), jnp.float32),
            pltpu.VMEM((4, te, 8, dp), jnp.float32),
            pltpu.SemaphoreType.DMA,
            pltpu.SemaphoreType.DMA,
            pltpu.SemaphoreType.DMA,
            pltpu.SemaphoreType.DMA,
            pltpu.SemaphoreType.DMA,
            pltpu.SemaphoreType.DMA,
            pltpu.SemaphoreType.DMA,
            pltpu.SemaphoreType.DMA,
        ],
    )
    def k(table_hbm, idx_hbm, out_hbm, idx_blk, idx_list, rows_p, panel,
          gsem0, gsem1, gsem2, gsem3, wsem0, wsem1, wsem2, wsem3):
        wid = lax.axis_index("s") * nc + lax.axis_index("c")
        tb0 = wid * tb_per_w
        gsem = (gsem0, gsem1, gsem2, gsem3)
        wsem = (wsem0, wsem1, wsem2, wsem3)

        def load_blk(tb_local):
            pltpu.sync_copy(
                idx_hbm.at[pl.ds((tb0 + tb_local) * dp, dp)], idx_blk)

        def build_idx(hh, slot):
            lane = lax.iota(jnp.int32, _L)
            hv = jnp.full((_L,), hh, jnp.int32)
            for j0 in range(0, dp, _L):
                vals = plsc.load_gather(idx_blk, [j0 + lane, hv])
                idx_list[slot, pl.ds(j0, _L)] = vals

        def start_gather(slot):
            pltpu.make_async_copy(
                table_hbm.at[idx_list.at[slot]],
                rows_p.at[slot],
                gsem[slot],
            ).start()

        def wait_gather(slot):
            pltpu.make_async_copy(
                table_hbm.at[idx_list.at[slot]],
                rows_p.at[slot],
                gsem[slot],
            ).wait()

        def transpose(slot):
            src = rows_p.at[slot]

            def body(i, carry):
                lane = lax.iota(jnp.int32, _L)
                for t in range(te):
                    ev = jnp.full((_L,), 8 * t + i, jnp.int32)
                    for j0 in range(0, dp, _L):
                        vals = plsc.load_gather(src, [j0 + lane, ev])
                        panel[slot, t, i, pl.ds(j0, _L)] = vals
                return carry

            lax.fori_loop(0, 8, body, 0)

        def start_write(hh, tb_local, slot):
            pltpu.make_async_copy(
                panel.at[slot],
                out_hbm.at[hh, :, tb0 + tb_local],
                wsem[slot],
            ).start()

        def wait_write(slot):
            pltpu.make_async_copy(
                panel.at[slot],
                out_hbm.at[0, :, 0],
                wsem[slot],
            ).wait()

        def hh_of(p):
            return lax.rem(p, h)

        def tb_of(p):
            return lax.div(p, h)

        load_blk(0)
        for u in range(3):
            build_idx(jnp.int32(u), u)
            start_gather(u)

        def step(p_dyn, u, guard):
            pn = p_dyn + 3

            @pl.when(jnp.logical_and(pn < npan, hh_of(pn) == 0))
            def _():
                load_blk(tb_of(pn))

            @pl.when(pn < npan)
            def _():
                build_idx(hh_of(pn), (u + 3) % 4)
                start_gather((u + 3) % 4)

            if guard:
                @pl.when(p_dyn >= 4)
                def _():
                    wait_write(u)
            else:
                wait_write(u)

            wait_gather(u)
            transpose(u)
            start_write(hh_of(p_dyn), tb_of(p_dyn), u)

        def body(i, carry):
            for u in range(4):
                step(4 * i + u, u, False)
            return carry

        for u in range(4):
            step(jnp.int32(u), u, True)
        lax.fori_loop(1, npan // 4, body, 0)
        for u in range(4):
            wait_write(u)

    return k(vp, idx)


def kernel(v, idx):
    b, h = idx.shape
    d = v.shape[1]
    vp = jnp.pad(v, ((0, 0), (0, 128 - d)))
    y5 = _gather_sc(vp, idx, d)
    return y5.transpose(2, 4, 0, 1, 3).reshape(b, h, d)


# speedup vs baseline: 1.7812x; 1.7780x over previous
import functools

import jax
import jax.numpy as jnp
from jax import lax
from jax.experimental import pallas as pl
from jax.experimental.pallas import tpu as pltpu
from jax.experimental.pallas import tpu_sc as plsc

_NB = 4
_NS = 4


@functools.partial(jax.jit, static_argnames=("hp", "nb"))
def _gather_sc(vp, idx, hp, nb):
    b, h = idx.shape
    dp = vp.shape[1]
    info = plsc.get_sparse_core_info()
    nc = info.num_cores
    nw = nc * info.num_subcores
    rows_per_w = b // nw
    n_chunks = rows_per_w // nb

    mesh = plsc.VectorSubcoreMesh(core_axis_name="c", subcore_axis_name="s")

    @functools.partial(
        pl.kernel,
        mesh=mesh,
        out_type=jax.ShapeDtypeStruct((b, hp, dp), jnp.float32),
        compiler_params=pltpu.CompilerParams(use_tc_tiling_on_sc=False,
                                             needs_layout_passes=False),
        scratch_types=[
            pltpu.VMEM((rows_per_w, h), jnp.int32),
            pltpu.VMEM((_NS, nb, h, dp), jnp.float32),
            pltpu.SemaphoreType.DMA,
            pltpu.SemaphoreType.DMA,
            pltpu.SemaphoreType.DMA,
            pltpu.SemaphoreType.DMA,
            pltpu.SemaphoreType.DMA,
            pltpu.SemaphoreType.DMA,
            pltpu.SemaphoreType.DMA,
            pltpu.SemaphoreType.DMA,
        ],
    )
    def k(table_hbm, idx_hbm, out_hbm, idx_all, rows_v,
          gsem0, gsem1, gsem2, gsem3, wsem0, wsem1, wsem2, wsem3):
        wid = lax.axis_index("s") * nc + lax.axis_index("c")
        base = wid * rows_per_w
        gsem = (gsem0, gsem1, gsem2, gsem3)
        wsem = (wsem0, wsem1, wsem2, wsem3)

        pltpu.sync_copy(idx_hbm.at[pl.ds(base, rows_per_w)], idx_all)

        def start_gathers(g, slot):
            for j in range(nb):
                pltpu.make_async_copy(
                    table_hbm.at[idx_all.at[g * nb + j]],
                    rows_v.at[slot, j],
                    gsem[slot],
                ).start()

        def wait_gathers(slot):
            pltpu.make_async_copy(
                table_hbm.at[idx_all.at[0]],
                rows_v.at[slot],
                gsem[slot],
            ).wait()

        def start_write(g, slot):
            pltpu.make_async_copy(
                rows_v.at[slot],
                out_hbm.at[pl.ds(base + g * nb, nb), pl.ds(0, h), :],
                wsem[slot],
            ).start()

        def wait_write(slot):
            pltpu.make_async_copy(
                rows_v.at[slot],
                out_hbm.at[pl.ds(base, nb), pl.ds(0, h), :],
                wsem[slot],
            ).wait()

        for u in range(_NS - 1):
            start_gathers(u, u)

        def step(g_dyn, u):
            s3 = (u + 3) % _NS
            pn = g_dyn + 3

            @pl.when(jnp.logical_and(pn < n_chunks, g_dyn >= 1))
            def _():
                wait_write(s3)

            @pl.when(pn < n_chunks)
            def _():
                start_gathers(pn, s3)

            wait_gathers(u)
            start_write(g_dyn, u)

        def body(i, carry):
            for u in range(_NS):
                step(_NS * i + u, u)
            return carry

        lax.fori_loop(0, n_chunks // _NS, body, 0)
        for u in range(_NS):
            wait_write(u)

    return k(vp, idx)


def kernel(v, idx):
    b, h = idx.shape
    d = v.shape[1]
    hp = 56
    vp = jnp.pad(v, ((0, 0), (0, 128 - d)))
    y = _gather_sc(vp, idx, hp, _NB)
    return y[:, :h, :d]


# speedup vs baseline: 1.7831x; 1.0011x over previous
import functools

import jax
import jax.numpy as jnp
from jax import lax
from jax.experimental import pallas as pl
from jax.experimental.pallas import tpu as pltpu
from jax.experimental.pallas import tpu_sc as plsc

_NB = 2
_NS = 8


@functools.partial(jax.jit, static_argnames=("hp", "nb"))
def _gather_sc(vp, idx, hp, nb):
    b, h = idx.shape
    dp = vp.shape[1]
    info = plsc.get_sparse_core_info()
    nc = info.num_cores
    nw = nc * info.num_subcores
    rows_per_w = b // nw
    n_chunks = rows_per_w // nb

    mesh = plsc.VectorSubcoreMesh(core_axis_name="c", subcore_axis_name="s")

    @functools.partial(
        pl.kernel,
        mesh=mesh,
        out_type=jax.ShapeDtypeStruct((b, hp, dp), jnp.float32),
        compiler_params=pltpu.CompilerParams(use_tc_tiling_on_sc=False,
                                             needs_layout_passes=False),
        scratch_types=[
            pltpu.VMEM((rows_per_w, h), jnp.int32),
            pltpu.VMEM((_NS, nb, h, dp), jnp.float32),
        ] + [pltpu.SemaphoreType.DMA] * (2 * _NS),
    )
    def k(table_hbm, idx_hbm, out_hbm, idx_all, rows_v, *sems):
        wid = lax.axis_index("s") * nc + lax.axis_index("c")
        base = wid * rows_per_w
        gsem = sems[:_NS]
        wsem = sems[_NS:]

        pltpu.sync_copy(idx_hbm.at[pl.ds(base, rows_per_w)], idx_all)

        def start_gathers(g, slot):
            for j in range(nb):
                pltpu.make_async_copy(
                    table_hbm.at[idx_all.at[g * nb + j]],
                    rows_v.at[slot, j],
                    gsem[slot],
                ).start()

        def wait_gathers(slot):
            pltpu.make_async_copy(
                table_hbm.at[idx_all.at[0]],
                rows_v.at[slot],
                gsem[slot],
            ).wait()

        def start_write(g, slot):
            pltpu.make_async_copy(
                rows_v.at[slot],
                out_hbm.at[pl.ds(base + g * nb, nb), pl.ds(0, h), :],
                wsem[slot],
            ).start()

        def wait_write(slot):
            pltpu.make_async_copy(
                rows_v.at[slot],
                out_hbm.at[pl.ds(base, nb), pl.ds(0, h), :],
                wsem[slot],
            ).wait()

        for u in range(_NS - 1):
            start_gathers(u, u)

        def step(g_dyn, u):
            s3 = (u + _NS - 1) % _NS
            pn = g_dyn + _NS - 1

            @pl.when(jnp.logical_and(pn < n_chunks, g_dyn >= 1))
            def _():
                wait_write(s3)

            @pl.when(pn < n_chunks)
            def _():
                start_gathers(pn, s3)

            wait_gathers(u)
            start_write(g_dyn, u)

        def body(i, carry):
            for u in range(_NS):
                step(_NS * i + u, u)
            return carry

        lax.fori_loop(0, n_chunks // _NS, body, 0)
        for u in range(_NS):
            wait_write(u)

    return k(vp, idx)


def kernel(v, idx):
    b, h = idx.shape
    d = v.shape[1]
    hp = 56
    vp = jnp.pad(v, ((0, 0), (0, 128 - d)))
    y = _gather_sc(vp, idx, hp, _NB)
    return y[:, :h, :d]
